# baseline (speedup 1.0000x reference)
import functools

import numpy as np
import jax
import jax.numpy as jnp
from jax import lax
from jax.experimental import pallas as pl
from jax.experimental.pallas import tpu as pltpu

_KSUB = 10
_NEG = np.float32(-300.0)
_CLIP = np.float32(-1e-7)
_LN2 = np.float32(-0.6931471805599453)


def _rotl(x, r):
    return ((x << np.uint32(r)) | (x >> np.uint32(32 - r))).astype(np.uint32)


def _threefry2x32(k0, k1, x0, x1):
    rotations = (13, 15, 26, 6, 17, 29, 16, 24)
    ks0 = np.uint32(k0)
    ks1 = np.uint32(k1)
    ks2 = np.uint32(ks0 ^ ks1 ^ np.uint32(0x1BD11BDA))
    ks = (ks0, ks1, ks2)
    x0 = (x0 + ks0).astype(np.uint32)
    x1 = (x1 + ks1).astype(np.uint32)
    for i in range(5):
        rots = rotations[0:4] if i % 2 == 0 else rotations[4:8]
        for r in rots:
            x0 = (x0 + x1).astype(np.uint32)
            x1 = _rotl(x1, r)
            x1 = (x1 ^ x0).astype(np.uint32)
        x0 = (x0 + ks[(i + 1) % 3]).astype(np.uint32)
        x1 = (x1 + ks[(i + 2) % 3] + np.uint32(i + 1)).astype(np.uint32)
    return x0, x1


@functools.lru_cache(maxsize=None)
def _uniforms_by_pos(n, b):
    key = np.array([0, 42], dtype=np.uint32)
    zeros2 = np.zeros(2, np.uint32)
    count2 = np.arange(2, dtype=np.uint32)
    zerosb = np.zeros(b, np.uint32)
    countb = np.arange(b, dtype=np.uint32)
    out = np.empty((n, b), dtype=np.float32)
    for t in range(n):
        o0, o1 = _threefry2x32(key[0], key[1], zeros2, count2)
        key = np.array([o0[0], o1[0]], np.uint32)
        s0, s1 = _threefry2x32(o0[1], o1[1], zerosb, countb)
        bits = (s0 ^ s1).astype(np.uint32)
        fb = ((bits >> np.uint32(9)) | np.uint32(0x3F800000)).astype(np.uint32)
        out[t] = fb.view(np.float32) - np.float32(1.0)
    return np.flipud(out).copy()


def _log1mexp(x):
    big = x > _LN2
    e = jnp.exp(x)
    return jnp.where(
        big,
        jnp.log(-(e - np.float32(1.0))),
        jnp.log1p(-e),
    )


def _logaddexp(x1, x2):
    amax = jnp.maximum(x1, x2)
    d = x1 - x2
    return amax + jnp.log1p(jnp.exp(-jnp.abs(d)))


def _body(theta_ref, u_ref, out_ref, lp_ref, lq_ref, t_ref):
    n, bb = theta_ref.shape
    kk2 = _KSUB + 2

    th = theta_ref[...]
    negth = -th
    softplus = jnp.maximum(negth, np.float32(0.0)) + jnp.log1p(
        jnp.exp(-jnp.abs(negth))
    )
    lp = jnp.minimum(-softplus, _CLIP)
    lq = _log1mexp(lp)
    lp_ref[...] = lp
    lq_ref[...] = lq

    iota_k = lax.broadcasted_iota(jnp.int32, (kk2, bb), 0)
    state0 = jnp.where(iota_k == 1, np.float32(0.0), _NEG)

    def dp_math(state, i):
        lp_i = lp_ref[pl.ds(i, 1), :]
        lq_i = lq_ref[pl.ds(i, 1), :]
        new = _logaddexp(state[: kk2 - 1] + lp_i, state[1:] + lq_i)
        return jnp.concatenate(
            [jnp.full((1, bb), _NEG, jnp.float32), new], axis=0
        )

    def table_row(sp, s, m):
        sp_sh = jnp.concatenate([sp[0:1], sp[0 : kk2 - 1]], axis=0)
        lp_m = lp_ref[pl.ds(m, 1), :]
        praw = jnp.minimum((sp_sh + lp_m) - s, _CLIP)
        q = _log1mexp(praw)
        t_ref[m] = jax.nn.sigmoid(praw - q)

    state1 = dp_math(state0, 0)

    def dp_step(i, carry):
        sp, s = carry
        new = dp_math(s, i)
        table_row(sp, s, i - 1)
        return (s, new)

    sp, s = lax.fori_loop(1, n, dp_step, (state0, state1))
    table_row(sp, s, n - 1)

    def s_step(t, j):
        m = n - 1 - t
        probrow = t_ref[m]
        mz = iota_k == jnp.maximum(j, 0)
        prob = jnp.sum(
            jnp.where(mz, probrow, np.float32(0.0)), axis=0, keepdims=True
        )
        xb = u_ref[pl.ds(m, 1), :] < prob
        out_ref[pl.ds(m, 1), :] = jnp.where(
            xb, np.float32(1.0), np.float32(0.0)
        )
        return jnp.where(xb, j - 1, j)

    j0 = jnp.full((1, bb), kk2 - 1, jnp.int32)
    lax.fori_loop(0, n, s_step, j0)


def _sample_call(theta_t, u):
    n, b = theta_t.shape
    kk2 = _KSUB + 2
    return pl.pallas_call(
        _body,
        out_shape=jax.ShapeDtypeStruct((n, b), jnp.float32),
        scratch_shapes=[
            pltpu.VMEM((n, b), jnp.float32),
            pltpu.VMEM((n, b), jnp.float32),
            pltpu.VMEM((n, kk2, b), jnp.float32),
        ],
    )(theta_t, u)


def kernel(logits):
    theta = jnp.squeeze(logits, -1)
    b, n = theta.shape
    u = jnp.asarray(_uniforms_by_pos(n, b))
    samples_t = _sample_call(theta.T, u)
    return samples_t.T[..., None]


# speedup vs baseline: 1.1605x; 1.1605x over previous
import functools

import numpy as np
import jax
import jax.numpy as jnp
from jax import lax
from jax.experimental import pallas as pl
from jax.experimental.pallas import tpu as pltpu

_KSUB = 10
_NEG = np.float32(-300.0)
_CLIP = np.float32(-1e-7)
_LN2 = np.float32(-0.6931471805599453)


def _rotl(x, r):
    return ((x << np.uint32(r)) | (x >> np.uint32(32 - r))).astype(np.uint32)


def _threefry2x32(k0, k1, x0, x1):
    rotations = (13, 15, 26, 6, 17, 29, 16, 24)
    ks0 = np.uint32(k0)
    ks1 = np.uint32(k1)
    ks2 = np.uint32(ks0 ^ ks1 ^ np.uint32(0x1BD11BDA))
    ks = (ks0, ks1, ks2)
    x0 = (x0 + ks0).astype(np.uint32)
    x1 = (x1 + ks1).astype(np.uint32)
    for i in range(5):
        rots = rotations[0:4] if i % 2 == 0 else rotations[4:8]
        for r in rots:
            x0 = (x0 + x1).astype(np.uint32)
            x1 = _rotl(x1, r)
            x1 = (x1 ^ x0).astype(np.uint32)
        x0 = (x0 + ks[(i + 1) % 3]).astype(np.uint32)
        x1 = (x1 + ks[(i + 2) % 3] + np.uint32(i + 1)).astype(np.uint32)
    return x0, x1


@functools.lru_cache(maxsize=None)
def _uniforms_by_pos(n, b):
    key = np.array([0, 42], dtype=np.uint32)
    zeros2 = np.zeros(2, np.uint32)
    count2 = np.arange(2, dtype=np.uint32)
    zerosb = np.zeros(b, np.uint32)
    countb = np.arange(b, dtype=np.uint32)
    out = np.empty((n, b), dtype=np.float32)
    for t in range(n):
        o0, o1 = _threefry2x32(key[0], key[1], zeros2, count2)
        key = np.array([o0[0], o1[0]], np.uint32)
        s0, s1 = _threefry2x32(o0[1], o1[1], zerosb, countb)
        bits = (s0 ^ s1).astype(np.uint32)
        fb = ((bits >> np.uint32(9)) | np.uint32(0x3F800000)).astype(np.uint32)
        out[t] = fb.view(np.float32) - np.float32(1.0)
    return np.flipud(out).copy()


def _log1mexp(x):
    big = x > _LN2
    e = jnp.exp(x)
    return jnp.where(
        big,
        jnp.log(-(e - np.float32(1.0))),
        jnp.log1p(-e),
    )


def _logaddexp(x1, x2):
    amax = jnp.maximum(x1, x2)
    d = x1 - x2
    return amax + jnp.log1p(jnp.exp(-jnp.abs(d)))


def _body(theta_ref, u_ref, out_ref, a_ref, lp_ref, lq_ref, t_ref, o_ref):
    bb, n = theta_ref.shape
    kk2 = _KSUB + 2

    th = jnp.transpose(theta_ref[...])
    negth = -th
    softplus = jnp.maximum(negth, np.float32(0.0)) + jnp.log1p(
        jnp.exp(-jnp.abs(negth))
    )
    lp = jnp.minimum(-softplus, _CLIP)
    lq = _log1mexp(lp)
    lp_ref[...] = lp
    lq_ref[...] = lq

    iota_k = lax.broadcasted_iota(jnp.int32, (kk2, bb), 0)
    state0 = jnp.where(iota_k == 1, np.float32(0.0), _NEG)
    a_ref[0] = state0

    def dp_step(i, state):
        lp_i = lp_ref[pl.ds(i, 1), :]
        lq_i = lq_ref[pl.ds(i, 1), :]
        new = _logaddexp(state[: kk2 - 1] + lp_i, state[1:] + lq_i)
        state = jnp.concatenate(
            [jnp.full((1, bb), _NEG, jnp.float32), new], axis=0
        )
        a_ref[i + 1] = state
        return state

    lax.fori_loop(0, n, dp_step, state0)

    a_prev = a_ref[0:n]
    a_cur = a_ref[1 : n + 1]
    a_prev_sh = jnp.concatenate(
        [a_prev[:, 0:1, :], a_prev[:, 0 : kk2 - 1, :]], axis=1
    )
    lp3 = lp_ref[...].reshape(n, 1, bb)
    praw = jnp.minimum((a_prev_sh + lp3) - a_cur, _CLIP)
    q = _log1mexp(praw)
    t_ref[...] = jax.nn.sigmoid(praw - q)

    def s_step(t, j):
        m = n - 1 - t
        probrow = t_ref[m]
        mz = iota_k == jnp.maximum(j, 0)
        prob = jnp.sum(
            jnp.where(mz, probrow, np.float32(0.0)), axis=0, keepdims=True
        )
        xb = u_ref[pl.ds(m, 1), :] < prob
        o_ref[pl.ds(m, 1), :] = jnp.where(
            xb, np.float32(1.0), np.float32(0.0)
        )
        return jnp.where(xb, j - 1, j)

    j0 = jnp.full((1, bb), kk2 - 1, jnp.int32)
    lax.fori_loop(0, n, s_step, j0)

    out_ref[...] = jnp.transpose(o_ref[...])


def _sample_call(theta, u):
    b, n = theta.shape
    kk2 = _KSUB + 2
    return pl.pallas_call(
        _body,
        out_shape=jax.ShapeDtypeStruct((b, n), jnp.float32),
        scratch_shapes=[
            pltpu.VMEM((n + 1, kk2, b), jnp.float32),
            pltpu.VMEM((n, b), jnp.float32),
            pltpu.VMEM((n, b), jnp.float32),
            pltpu.VMEM((n, kk2, b), jnp.float32),
            pltpu.VMEM((n, b), jnp.float32),
        ],
    )(theta, u)


def kernel(logits):
    theta = jnp.squeeze(logits, -1)
    b, n = theta.shape
    u = jnp.asarray(_uniforms_by_pos(n, b))
    samples = _sample_call(theta, u)
    return samples[..., None]


# speedup vs baseline: 1.5775x; 1.3594x over previous
import functools

import numpy as np
import jax
import jax.numpy as jnp
from jax import lax
from jax.experimental import pallas as pl
from jax.experimental.pallas import tpu as pltpu

_KSUB = 10
_NEG = np.float32(-300.0)
_CLIP = np.float32(-1e-7)
_LN2 = np.float32(-0.6931471805599453)


def _rotl(x, r):
    return ((x << np.uint32(r)) | (x >> np.uint32(32 - r))).astype(np.uint32)


def _threefry2x32(k0, k1, x0, x1):
    rotations = (13, 15, 26, 6, 17, 29, 16, 24)
    ks0 = np.uint32(k0)
    ks1 = np.uint32(k1)
    ks2 = np.uint32(ks0 ^ ks1 ^ np.uint32(0x1BD11BDA))
    ks = (ks0, ks1, ks2)
    x0 = (x0 + ks0).astype(np.uint32)
    x1 = (x1 + ks1).astype(np.uint32)
    for i in range(5):
        rots = rotations[0:4] if i % 2 == 0 else rotations[4:8]
        for r in rots:
            x0 = (x0 + x1).astype(np.uint32)
            x1 = _rotl(x1, r)
            x1 = (x1 ^ x0).astype(np.uint32)
        x0 = (x0 + ks[(i + 1) % 3]).astype(np.uint32)
        x1 = (x1 + ks[(i + 2) % 3] + np.uint32(i + 1)).astype(np.uint32)
    return x0, x1


@functools.lru_cache(maxsize=None)
def _uniforms_by_pos(n, b):
    key = np.array([0, 42], dtype=np.uint32)
    zeros2 = np.zeros(2, np.uint32)
    count2 = np.arange(2, dtype=np.uint32)
    zerosb = np.zeros(b, np.uint32)
    countb = np.arange(b, dtype=np.uint32)
    out = np.empty((n, b), dtype=np.float32)
    for t in range(n):
        o0, o1 = _threefry2x32(key[0], key[1], zeros2, count2)
        key = np.array([o0[0], o1[0]], np.uint32)
        s0, s1 = _threefry2x32(o0[1], o1[1], zerosb, countb)
        bits = (s0 ^ s1).astype(np.uint32)
        fb = ((bits >> np.uint32(9)) | np.uint32(0x3F800000)).astype(np.uint32)
        out[t] = fb.view(np.float32) - np.float32(1.0)
    return np.flipud(out).copy()


def _log1mexp(x):
    big = x > _LN2
    e = jnp.exp(x)
    return jnp.where(
        big,
        jnp.log(-(e - np.float32(1.0))),
        jnp.log1p(-e),
    )


def _logaddexp(x1, x2):
    amax = jnp.maximum(x1, x2)
    d = x1 - x2
    return amax + jnp.log1p(jnp.exp(-jnp.abs(d)))


def _body(theta_ref, u_ref, out_ref, a_ref, lp_ref, lq_ref, t_ref, o_ref):
    bb, n = theta_ref.shape
    kk2 = _KSUB + 2

    th = jnp.transpose(theta_ref[...])
    negth = -th
    softplus = jnp.maximum(negth, np.float32(0.0)) + jnp.log1p(
        jnp.exp(-jnp.abs(negth))
    )
    lp = jnp.minimum(-softplus, _CLIP)
    lq = _log1mexp(lp)
    lp_ref[...] = lp
    lq_ref[...] = lq

    iota_k = lax.broadcasted_iota(jnp.int32, (kk2, bb), 0)
    state0 = jnp.where(iota_k == 1, np.float32(0.0), _NEG)
    a_ref[0] = state0

    def dp_step(i, state):
        lp_i = lp_ref[pl.ds(i, 1), :]
        lq_i = lq_ref[pl.ds(i, 1), :]
        new = _logaddexp(state[: kk2 - 1] + lp_i, state[1:] + lq_i)
        state = jnp.concatenate(
            [jnp.full((1, bb), _NEG, jnp.float32), new], axis=0
        )
        a_ref[i + 1] = state
        return state

    if n % 2 == 0:

        def dp_pair(t, state):
            state = dp_step(2 * t, state)
            return dp_step(2 * t + 1, state)

        lax.fori_loop(0, n // 2, dp_pair, state0)
    else:
        lax.fori_loop(0, n, dp_step, state0)

    a_prev = a_ref[0:n]
    a_cur = a_ref[1 : n + 1]
    a_prev_sh = jnp.concatenate(
        [a_prev[:, 0:1, :], a_prev[:, 0 : kk2 - 1, :]], axis=1
    )
    lp3 = lp_ref[...].reshape(n, 1, bb)
    praw = jnp.minimum((a_prev_sh + lp3) - a_cur, _CLIP)
    q = _log1mexp(praw)
    t_ref[...] = jax.nn.sigmoid(praw - q)

    o_ref[...] = jnp.where(
        u_ref[...] < t_ref[:, 0, :], np.float32(1.0), np.float32(0.0)
    )

    def s_step(t, j):
        m = n - 1 - t
        probrow = t_ref[m]
        mz = iota_k == jnp.maximum(j, 0)
        prob = jnp.sum(
            jnp.where(mz, probrow, np.float32(0.0)), axis=0, keepdims=True
        )
        xb = u_ref[pl.ds(m, 1), :] < prob
        o_ref[pl.ds(m, 1), :] = jnp.where(
            xb, np.float32(1.0), np.float32(0.0)
        )
        return jnp.where(xb, j - 1, j)

    blk = 64 if n % 64 == 0 else 1
    j0 = jnp.full((1, bb), kk2 - 1, jnp.int32)

    def blk_cond(carry):
        t0, j = carry
        return jnp.logical_and(t0 < n, jnp.max(j) > 0)

    def blk_body(carry):
        t0, j = carry
        j = lax.fori_loop(0, blk, lambda k, jj: s_step(t0 + k, jj), j)
        return (t0 + blk, j)

    lax.while_loop(blk_cond, blk_body, (0, j0))

    out_ref[...] = jnp.transpose(o_ref[...])


def _sample_call(theta, u):
    b, n = theta.shape
    kk2 = _KSUB + 2
    return pl.pallas_call(
        _body,
        out_shape=jax.ShapeDtypeStruct((b, n), jnp.float32),
        scratch_shapes=[
            pltpu.VMEM((n + 1, kk2, b), jnp.float32),
            pltpu.VMEM((n, b), jnp.float32),
            pltpu.VMEM((n, b), jnp.float32),
            pltpu.VMEM((n, kk2, b), jnp.float32),
            pltpu.VMEM((n, b), jnp.float32),
        ],
    )(theta, u)


def kernel(logits):
    theta = jnp.squeeze(logits, -1)
    b, n = theta.shape
    u = jnp.asarray(_uniforms_by_pos(n, b))
    samples = _sample_call(theta, u)
    return samples[..., None]


# speedup vs baseline: 1.9711x; 1.2496x over previous
import functools

import numpy as np
import jax
import jax.numpy as jnp
from jax import lax
from jax.experimental import pallas as pl
from jax.experimental.pallas import tpu as pltpu

_KSUB = 10
_NEG = np.float32(-300.0)
_CLIP = np.float32(-1e-7)
_LN2 = np.float32(-0.6931471805599453)


def _rotl(x, r):
    return ((x << np.uint32(r)) | (x >> np.uint32(32 - r))).astype(np.uint32)


def _threefry2x32(k0, k1, x0, x1):
    rotations = (13, 15, 26, 6, 17, 29, 16, 24)
    ks0 = np.uint32(k0)
    ks1 = np.uint32(k1)
    ks2 = np.uint32(ks0 ^ ks1 ^ np.uint32(0x1BD11BDA))
    ks = (ks0, ks1, ks2)
    x0 = (x0 + ks0).astype(np.uint32)
    x1 = (x1 + ks1).astype(np.uint32)
    for i in range(5):
        rots = rotations[0:4] if i % 2 == 0 else rotations[4:8]
        for r in rots:
            x0 = (x0 + x1).astype(np.uint32)
            x1 = _rotl(x1, r)
            x1 = (x1 ^ x0).astype(np.uint32)
        x0 = (x0 + ks[(i + 1) % 3]).astype(np.uint32)
        x1 = (x1 + ks[(i + 2) % 3] + np.uint32(i + 1)).astype(np.uint32)
    return x0, x1


@functools.lru_cache(maxsize=None)
def _uniforms_by_pos(n, b):
    key = np.array([0, 42], dtype=np.uint32)
    zeros2 = np.zeros(2, np.uint32)
    count2 = np.arange(2, dtype=np.uint32)
    zerosb = np.zeros(b, np.uint32)
    countb = np.arange(b, dtype=np.uint32)
    out = np.empty((n, b), dtype=np.float32)
    for t in range(n):
        o0, o1 = _threefry2x32(key[0], key[1], zeros2, count2)
        key = np.array([o0[0], o1[0]], np.uint32)
        s0, s1 = _threefry2x32(o0[1], o1[1], zerosb, countb)
        bits = (s0 ^ s1).astype(np.uint32)
        fb = ((bits >> np.uint32(9)) | np.uint32(0x3F800000)).astype(np.uint32)
        out[t] = fb.view(np.float32) - np.float32(1.0)
    return np.flipud(out).copy()


def _log1mexp(x):
    big = x > _LN2
    e = jnp.exp(x)
    return jnp.where(
        big,
        jnp.log(-(e - np.float32(1.0))),
        jnp.log1p(-e),
    )


def _logaddexp(x1, x2):
    amax = jnp.maximum(x1, x2)
    d = x1 - x2
    return amax + jnp.log1p(jnp.exp(-jnp.abs(d)))


def _body(theta_ref, u_ref, out_ref, a_ref, lp_ref, lq_ref, t_ref, o_ref):
    bb, n = theta_ref.shape
    kk2 = _KSUB + 2

    th = jnp.transpose(theta_ref[...])
    negth = -th
    softplus = jnp.maximum(negth, np.float32(0.0)) + jnp.log1p(
        jnp.exp(-jnp.abs(negth))
    )
    lp = jnp.minimum(-softplus, _CLIP)
    lq = _log1mexp(lp)
    lp_ref[...] = lp
    lq_ref[...] = lq

    iota_k = lax.broadcasted_iota(jnp.int32, (kk2, bb), 0)
    state0 = jnp.where(iota_k == 1, np.float32(0.0), _NEG)
    a_ref[0] = state0

    def dp_step(i, state):
        lp_i = lp_ref[pl.ds(i, 1), :]
        lq_i = lq_ref[pl.ds(i, 1), :]
        new = _logaddexp(state[: kk2 - 1] + lp_i, state[1:] + lq_i)
        state = jnp.concatenate(
            [jnp.full((1, bb), _NEG, jnp.float32), new], axis=0
        )
        a_ref[i + 1] = state
        return state

    if n % 2 == 0:

        def dp_pair(t, state):
            state = dp_step(2 * t, state)
            return dp_step(2 * t + 1, state)

        lax.fori_loop(0, n // 2, dp_pair, state0)
    else:
        lax.fori_loop(0, n, dp_step, state0)

    praw0 = jnp.minimum((_NEG + lp) - _NEG, _CLIP)
    q0 = _log1mexp(praw0)
    t0prob = jax.nn.sigmoid(praw0 - q0)
    o_ref[...] = jnp.where(
        u_ref[...] < t0prob, np.float32(1.0), np.float32(0.0)
    )

    def s_step(t, j):
        m = n - 1 - t
        probrow = t_ref[m]
        mz = iota_k == jnp.maximum(j, 0)
        prob = jnp.sum(
            jnp.where(mz, probrow, np.float32(0.0)), axis=0, keepdims=True
        )
        xb = u_ref[pl.ds(m, 1), :] < prob
        o_ref[pl.ds(m, 1), :] = jnp.where(
            xb, np.float32(1.0), np.float32(0.0)
        )
        return jnp.where(xb, j - 1, j)

    blk = 64 if n % 64 == 0 else 1
    j0 = jnp.full((1, bb), kk2 - 1, jnp.int32)

    def blk_cond(carry):
        t0, j = carry
        return jnp.logical_and(t0 < n, jnp.max(j) > 0)

    def blk_body(carry):
        t0, j = carry
        m_lo = n - t0 - blk
        ap = a_ref[pl.ds(m_lo, blk)]
        ac = a_ref[pl.ds(m_lo + 1, blk)]
        ap_sh = jnp.concatenate(
            [ap[:, 0:1, :], ap[:, 0 : kk2 - 1, :]], axis=1
        )
        lpc = lp_ref[pl.ds(m_lo, blk)].reshape(blk, 1, bb)
        praw = jnp.minimum((ap_sh + lpc) - ac, _CLIP)
        q = _log1mexp(praw)
        t_ref[pl.ds(m_lo, blk)] = jax.nn.sigmoid(praw - q)
        j = lax.fori_loop(0, blk, lambda k, jj: s_step(t0 + k, jj), j)
        return (t0 + blk, j)

    lax.while_loop(blk_cond, blk_body, (0, j0))

    out_ref[...] = jnp.transpose(o_ref[...])


def _sample_call(theta, u):
    b, n = theta.shape
    kk2 = _KSUB + 2
    return pl.pallas_call(
        _body,
        out_shape=jax.ShapeDtypeStruct((b, n), jnp.float32),
        scratch_shapes=[
            pltpu.VMEM((n + 1, kk2, b), jnp.float32),
            pltpu.VMEM((n, b), jnp.float32),
            pltpu.VMEM((n, b), jnp.float32),
            pltpu.VMEM((n, kk2, b), jnp.float32),
            pltpu.VMEM((n, b), jnp.float32),
        ],
    )(theta, u)


def kernel(logits):
    theta = jnp.squeeze(logits, -1)
    b, n = theta.shape
    u = jnp.asarray(_uniforms_by_pos(n, b))
    samples = _sample_call(theta, u)
    return samples[..., None]


# speedup vs baseline: 2.0244x; 1.0270x over previous
import functools

import numpy as np
import jax
import jax.numpy as jnp
from jax import lax
from jax.experimental import pallas as pl
from jax.experimental.pallas import tpu as pltpu

_KSUB = 10
_NEG = np.float32(-300.0)
_CLIP = np.float32(-1e-7)
_LN2 = np.float32(-0.6931471805599453)


def _rotl(x, r):
    return ((x << np.uint32(r)) | (x >> np.uint32(32 - r))).astype(np.uint32)


def _threefry2x32(k0, k1, x0, x1):
    rotations = (13, 15, 26, 6, 17, 29, 16, 24)
    ks0 = np.uint32(k0)
    ks1 = np.uint32(k1)
    ks2 = np.uint32(ks0 ^ ks1 ^ np.uint32(0x1BD11BDA))
    ks = (ks0, ks1, ks2)
    x0 = (x0 + ks0).astype(np.uint32)
    x1 = (x1 + ks1).astype(np.uint32)
    for i in range(5):
        rots = rotations[0:4] if i % 2 == 0 else rotations[4:8]
        for r in rots:
            x0 = (x0 + x1).astype(np.uint32)
            x1 = _rotl(x1, r)
            x1 = (x1 ^ x0).astype(np.uint32)
        x0 = (x0 + ks[(i + 1) % 3]).astype(np.uint32)
        x1 = (x1 + ks[(i + 2) % 3] + np.uint32(i + 1)).astype(np.uint32)
    return x0, x1


@functools.lru_cache(maxsize=None)
def _uniforms_by_pos(n, b):
    key = np.array([0, 42], dtype=np.uint32)
    zeros2 = np.zeros(2, np.uint32)
    count2 = np.arange(2, dtype=np.uint32)
    zerosb = np.zeros(b, np.uint32)
    countb = np.arange(b, dtype=np.uint32)
    out = np.empty((n, b), dtype=np.float32)
    for t in range(n):
        o0, o1 = _threefry2x32(key[0], key[1], zeros2, count2)
        key = np.array([o0[0], o1[0]], np.uint32)
        s0, s1 = _threefry2x32(o0[1], o1[1], zerosb, countb)
        bits = (s0 ^ s1).astype(np.uint32)
        fb = ((bits >> np.uint32(9)) | np.uint32(0x3F800000)).astype(np.uint32)
        out[t] = fb.view(np.float32) - np.float32(1.0)
    return np.flipud(out).copy()


def _log1mexp(x):
    big = x > _LN2
    e = jnp.exp(x)
    return jnp.where(
        big,
        jnp.log(-(e - np.float32(1.0))),
        jnp.log1p(-e),
    )


def _logaddexp(x1, x2):
    amax = jnp.maximum(x1, x2)
    d = x1 - x2
    return amax + jnp.log1p(jnp.exp(-jnp.abs(d)))


def _body(theta_ref, u_ref, out_ref, a_ref, lp_ref, lq_ref, t_ref, o_ref):
    bb, n = theta_ref.shape
    kk2 = _KSUB + 2

    th = jnp.transpose(theta_ref[...])
    negth = -th
    softplus = jnp.maximum(negth, np.float32(0.0)) + jnp.log1p(
        jnp.exp(-jnp.abs(negth))
    )
    lp = jnp.minimum(-softplus, _CLIP)
    lq = _log1mexp(lp)
    lp_ref[...] = lp
    lq_ref[...] = lq

    iota_k = lax.broadcasted_iota(jnp.int32, (kk2, bb), 0)
    state0 = jnp.where(iota_k == 1, np.float32(0.0), _NEG)
    a_ref[0] = state0

    def dp_step(i, state):
        lp_i = lp_ref[pl.ds(i, 1), :]
        lq_i = lq_ref[pl.ds(i, 1), :]
        new = _logaddexp(state[: kk2 - 1] + lp_i, state[1:] + lq_i)
        state = jnp.concatenate(
            [jnp.full((1, bb), _NEG, jnp.float32), new], axis=0
        )
        a_ref[i + 1] = state
        return state

    if n % 4 == 0:

        def dp_quad(t, state):
            state = dp_step(4 * t, state)
            state = dp_step(4 * t + 1, state)
            state = dp_step(4 * t + 2, state)
            return dp_step(4 * t + 3, state)

        lax.fori_loop(0, n // 4, dp_quad, state0)
    else:
        lax.fori_loop(0, n, dp_step, state0)

    praw0 = jnp.minimum((_NEG + lp) - _NEG, _CLIP)
    q0 = _log1mexp(praw0)
    t0prob = jax.nn.sigmoid(praw0 - q0)
    o_ref[...] = jnp.where(
        u_ref[...] < t0prob, np.float32(1.0), np.float32(0.0)
    )

    def s_step(t, j):
        m = n - 1 - t
        probrow = t_ref[m]
        mz = iota_k == jnp.maximum(j, 0)
        prob = jnp.sum(
            jnp.where(mz, probrow, np.float32(0.0)), axis=0, keepdims=True
        )
        xb = u_ref[pl.ds(m, 1), :] < prob
        o_ref[pl.ds(m, 1), :] = jnp.where(
            xb, np.float32(1.0), np.float32(0.0)
        )
        return jnp.where(xb, j - 1, j)

    blk = 64 if n % 64 == 0 else 1
    j0 = jnp.full((1, bb), kk2 - 1, jnp.int32)

    def blk_cond(carry):
        t0, j = carry
        return jnp.logical_and(t0 < n, jnp.max(j) > 0)

    def blk_body(carry):
        t0, j = carry
        m_lo = n - t0 - blk
        ap = a_ref[pl.ds(m_lo, blk)]
        ac = a_ref[pl.ds(m_lo + 1, blk)]
        ap_sh = jnp.concatenate(
            [ap[:, 0:1, :], ap[:, 0 : kk2 - 1, :]], axis=1
        )
        lpc = lp_ref[pl.ds(m_lo, blk)].reshape(blk, 1, bb)
        praw = jnp.minimum((ap_sh + lpc) - ac, _CLIP)
        q = _log1mexp(praw)
        t_ref[pl.ds(m_lo, blk)] = jax.nn.sigmoid(praw - q)
        j = lax.fori_loop(0, blk, lambda k, jj: s_step(t0 + k, jj), j)
        return (t0 + blk, j)

    lax.while_loop(blk_cond, blk_body, (0, j0))

    out_ref[...] = jnp.transpose(o_ref[...])


def _sample_call(theta, u):
    b, n = theta.shape
    kk2 = _KSUB + 2
    return pl.pallas_call(
        _body,
        out_shape=jax.ShapeDtypeStruct((b, n), jnp.float32),
        scratch_shapes=[
            pltpu.VMEM((n + 1, kk2, b), jnp.float32),
            pltpu.VMEM((n, b), jnp.float32),
            pltpu.VMEM((n, b), jnp.float32),
            pltpu.VMEM((n, kk2, b), jnp.float32),
            pltpu.VMEM((n, b), jnp.float32),
        ],
    )(theta, u)


def kernel(logits):
    theta = jnp.squeeze(logits, -1)
    b, n = theta.shape
    u = jnp.asarray(_uniforms_by_pos(n, b))
    samples = _sample_call(theta, u)
    return samples[..., None]
